# baseline (speedup 1.0000x reference)
import jax
import jax.numpy as jnp
from jax import lax
from jax.experimental import pallas as pl
from jax.experimental.pallas import tpu as pltpu
from jax.experimental.pallas import tpu_sc as plsc

_INFO = plsc.get_sparse_core_info()
_NC, _NS, _L = _INFO.num_cores, _INFO.num_subcores, _INFO.num_lanes
_NW = _NC * _NS

_BATCH = 4
_SEQ = 8192
_V = 32768
_D = 64
_ROWS_PER_W = _D * 2 // _NW
_NVEC = _SEQ // _L


def _body(pid_hbm, cosT_hbm, sinT_hbm, ocos_hbm, osin_hbm,
          idx2, row0, row1, out2, sh_idx, sem_i, sem_r, sem_o):
    sid = lax.axis_index("s")
    wid = sid * _NC + lax.axis_index("c")
    d0 = wid * 2
    units = [(p, b) for p in range(2) for b in range(_BATCH)]
    nu = len(units)

    @pl.when(sid == 0)
    def _stage_ids():
        pltpu.sync_copy(pid_hbm, sh_idx)

    plsc.subcore_barrier()

    def idx_prefetch(u):
        b = units[u][1]
        return pltpu.async_copy(sh_idx.at[pl.ds(b * _SEQ, _SEQ)],
                                idx2.at[pl.ds((u % 2) * _SEQ, _SEQ)], sem_i)

    def stage_rows(tab):
        return (pltpu.async_copy(tab.at[d0], row0, sem_r),
                pltpu.async_copy(tab.at[d0 + 1], row1, sem_r))

    icp = [None] * nu
    ocp = [None] * nu
    icp[0] = idx_prefetch(0)
    rcp = stage_rows(cosT_hbm)
    for u in range(nu):
        p, b = units[u]
        if u + 1 < nu:
            icp[u + 1] = idx_prefetch(u + 1)
        if u == 0 or u == _BATCH:
            rcp[0].wait()
            rcp[1].wait()
        icp[u].wait()
        if u >= 2:
            for cp in ocp[u - 2]:
                cp.wait()
        ioff = (u % 2) * _SEQ
        ooff = (u % 2) * 2 * _SEQ

        @plsc.parallel_loop(0, _NVEC, unroll=16)
        def gather(j):
            iv = idx2[pl.ds(ioff + j * _L, _L)]
            out2[pl.ds(ooff + j * _L, _L)] = plsc.load_gather(row0, [iv])
            out2[pl.ds(ooff + _SEQ + j * _L, _L)] = plsc.load_gather(row1, [iv])

        if u == _BATCH - 1:
            rcp = stage_rows(sinT_hbm)
        out = ocos_hbm if p == 0 else osin_hbm
        ocp[u] = (
            pltpu.async_copy(out2.at[pl.ds(ooff, _SEQ)], out.at[b].at[d0], sem_o),
            pltpu.async_copy(out2.at[pl.ds(ooff + _SEQ, _SEQ)],
                             out.at[b].at[d0 + 1], sem_o),
        )
    for cp in ocp[nu - 2] + ocp[nu - 1]:
        cp.wait()


@jax.jit
def _rope_gather(pid_flat, cosT, sinT):
    mesh = plsc.VectorSubcoreMesh(core_axis_name="c", subcore_axis_name="s")
    k = pl.kernel(
        _body,
        out_type=[
            jax.ShapeDtypeStruct((_BATCH, _D, _SEQ), jnp.float32),
            jax.ShapeDtypeStruct((_BATCH, _D, _SEQ), jnp.float32),
        ],
        mesh=mesh,
        scratch_types=[
            pltpu.VMEM((2 * _SEQ,), jnp.int32),
            pltpu.VMEM((_V,), jnp.float32),
            pltpu.VMEM((_V,), jnp.float32),
            pltpu.VMEM((4 * _SEQ,), jnp.float32),
            pltpu.VMEM_SHARED((_BATCH * _SEQ,), jnp.int32),
            pltpu.SemaphoreType.DMA,
            pltpu.SemaphoreType.DMA,
            pltpu.SemaphoreType.DMA,
        ],
        compiler_params=pltpu.CompilerParams(needs_layout_passes=False),
    )
    return k(pid_flat, cosT, sinT)


def kernel(x, position_ids, cos_cached, sin_cached):
    b, s = position_ids.shape
    pid = position_ids.reshape(b * s)
    ocos, osin = _rope_gather(pid, cos_cached.T, sin_cached.T)
    cos = jnp.swapaxes(ocos, 1, 2).astype(x.dtype)
    sin = jnp.swapaxes(osin, 1, 2).astype(x.dtype)
    return (cos, sin)


# speedup vs baseline: 1.0288x; 1.0288x over previous
import jax
import jax.numpy as jnp
from jax import lax
from jax.experimental import pallas as pl
from jax.experimental.pallas import tpu as pltpu
from jax.experimental.pallas import tpu_sc as plsc

_INFO = plsc.get_sparse_core_info()
_NC, _NS, _L = _INFO.num_cores, _INFO.num_subcores, _INFO.num_lanes
_NW = _NC * _NS

_BATCH = 4
_SEQ = 8192
_V = 32768
_D = 64
_ROWS_PER_W = _D * 2 // _NW
_NVEC = _SEQ // _L


def _body(pid_hbm, cosT_hbm, sinT_hbm, ocos_hbm, osin_hbm,
          idx2, row0, row1, out2, sh_idx, sem_i, sem_r, sem_o):
    sid = lax.axis_index("s")
    wid = sid * _NC + lax.axis_index("c")
    d0 = wid * 2
    units = [(p, b) for p in range(2) for b in range(_BATCH)]
    nu = len(units)

    @pl.when(sid == 0)
    def _stage_ids():
        pltpu.sync_copy(pid_hbm, sh_idx)

    plsc.subcore_barrier()

    def idx_prefetch(u):
        b = units[u][1]
        return pltpu.async_copy(sh_idx.at[pl.ds(b * _SEQ, _SEQ)],
                                idx2.at[pl.ds((u % 2) * _SEQ, _SEQ)], sem_i)

    def stage_rows(tab):
        return (pltpu.async_copy(tab.at[d0], row0, sem_r),
                pltpu.async_copy(tab.at[d0 + 1], row1, sem_r))

    icp = [None] * nu
    ocp = [None] * nu
    icp[0] = idx_prefetch(0)
    rcp = stage_rows(cosT_hbm)
    for u in range(nu):
        p, b = units[u]
        if u + 1 < nu:
            icp[u + 1] = idx_prefetch(u + 1)
        if u == 0 or u == _BATCH:
            rcp[0].wait()
            rcp[1].wait()
        icp[u].wait()
        if u >= 2:
            for cp in ocp[u - 2]:
                cp.wait()
        ioff = (u % 2) * _SEQ
        ooff = (u % 2) * 2 * _SEQ

        @plsc.parallel_loop(0, _NVEC, unroll=8)
        def gather(j):
            iv = idx2[pl.ds(ioff + j * _L, _L)]
            out2[pl.ds(ooff + j * _L, _L)] = plsc.load_gather(row0, [iv])
            out2[pl.ds(ooff + _SEQ + j * _L, _L)] = plsc.load_gather(row1, [iv])

        if u == _BATCH - 1:
            rcp = stage_rows(sinT_hbm)
        out = ocos_hbm if p == 0 else osin_hbm
        ocp[u] = (
            pltpu.async_copy(out2.at[pl.ds(ooff, _SEQ)], out.at[b].at[d0], sem_o),
            pltpu.async_copy(out2.at[pl.ds(ooff + _SEQ, _SEQ)],
                             out.at[b].at[d0 + 1], sem_o),
        )
    for cp in ocp[nu - 2] + ocp[nu - 1]:
        cp.wait()


@jax.jit
def _rope_gather(pid_flat, cosT, sinT):
    mesh = plsc.VectorSubcoreMesh(core_axis_name="c", subcore_axis_name="s")
    k = pl.kernel(
        _body,
        out_type=[
            jax.ShapeDtypeStruct((_BATCH, _D, _SEQ), jnp.float32),
            jax.ShapeDtypeStruct((_BATCH, _D, _SEQ), jnp.float32),
        ],
        mesh=mesh,
        scratch_types=[
            pltpu.VMEM((2 * _SEQ,), jnp.int32),
            pltpu.VMEM((_V,), jnp.float32),
            pltpu.VMEM((_V,), jnp.float32),
            pltpu.VMEM((4 * _SEQ,), jnp.float32),
            pltpu.VMEM_SHARED((_BATCH * _SEQ,), jnp.int32),
            pltpu.SemaphoreType.DMA,
            pltpu.SemaphoreType.DMA,
            pltpu.SemaphoreType.DMA,
        ],
        compiler_params=pltpu.CompilerParams(needs_layout_passes=False),
    )
    return k(pid_flat, cosT, sinT)


def kernel(x, position_ids, cos_cached, sin_cached):
    b, s = position_ids.shape
    pid = position_ids.reshape(b * s)
    ocos, osin = _rope_gather(pid, cos_cached.T, sin_cached.T)
    cos = jnp.swapaxes(ocos, 1, 2).astype(x.dtype)
    sin = jnp.swapaxes(osin, 1, 2).astype(x.dtype)
    return (cos, sin)
